# baseline (speedup 1.0000x reference)
import functools

import jax
import jax.numpy as jnp
from jax import lax
from jax.experimental import pallas as pl
from jax.experimental.pallas import tpu as pltpu
from jax.experimental.pallas import tpu_sc as plsc

E = 64
K = 2
D = 768
F = 1536
T = 128
L = 16
NCORE = 2
NSUB = 16
TG = T // L
Q = E // L
DCH = 16


def _gate_body(xg_hbm, wgq_hbm, bg_hbm, w_hbm,
               xv, wgv, bgv, stg, allv, exv, pv, wout, shared):
    cid = lax.axis_index("c")
    sid = lax.axis_index("s")
    tgl = sid // 4
    q = sid % 4
    tg = cid * 4 + tgl

    pltpu.sync_copy(xg_hbm.at[pl.ds(tg * D * L, D * L)], xv)
    pltpu.sync_copy(wgq_hbm.at[pl.ds(q * D * L, D * L)], wgv)
    pltpu.sync_copy(bg_hbm.at[pl.ds(q * L, L)], bgv)

    zero16 = jnp.zeros((L,), jnp.float32)

    def dstep(i, carry):
        part = [zero16] * L
        for j in range(DCH):
            d = i * DCH + j
            xrow = xv[pl.ds(d * L, L)]
            wrow = wgv[pl.ds(d * L, L)]
            for e in range(L):
                part[e] = part[e] + xrow * wrow[e]
        return tuple(c + p for c, p in zip(carry, part))

    init = tuple(zero16 for _ in range(L))
    lg = lax.fori_loop(0, D // DCH, dstep, init)

    bgrow = bgv[pl.ds(0, L)]
    for e in range(L):
        stg[pl.ds(e * L, L)] = lg[e] + bgrow[e]

    pltpu.sync_copy(stg, shared.at[pl.ds((tgl * 4 + q) * L * L, L * L)])
    plsc.subcore_barrier()

    @pl.when(q == 0)
    def _finalize():
        pltpu.sync_copy(shared.at[pl.ds(tgl * E * L, E * L)], allv)
        mv = allv[pl.ds(0, L)]
        for ge in range(1, E):
            mv = jnp.maximum(mv, allv[pl.ds(ge * L, L)])
        s = jnp.zeros((L,), jnp.float32)
        for ge in range(E):
            ex = jnp.exp(allv[pl.ds(ge * L, L)] - mv)
            exv[pl.ds(ge * L, L)] = ex
            s = s + ex
        m1 = jnp.full((L,), -1.0, jnp.float32)
        for ge in range(E):
            p = exv[pl.ds(ge * L, L)] / s
            pv[pl.ds(ge * L, L)] = p
            m1 = jnp.maximum(m1, p)
        big = jnp.full((L,), E, jnp.int32)
        i1 = big
        for ge in range(E):
            p = pv[pl.ds(ge * L, L)]
            i1 = jnp.minimum(i1, jnp.where(p == m1, ge, E))
        m2 = jnp.full((L,), -1.0, jnp.float32)
        for ge in range(E):
            p = pv[pl.ds(ge * L, L)]
            m2 = jnp.maximum(m2, jnp.where(i1 == ge, -1.0, p))
        i2 = big
        for ge in range(E):
            p = jnp.where(i1 == ge, -1.0, pv[pl.ds(ge * L, L)])
            i2 = jnp.minimum(i2, jnp.where(p == m2, ge, E))
        denom = m1 + m2
        a1 = m1 / denom
        a2 = m2 / denom
        for ge in range(E):
            wge = (jnp.where(i1 == ge, a1, 0.0)
                   + jnp.where(i2 == ge, a2, 0.0))
            wout[pl.ds(ge * L, L)] = wge
        pltpu.sync_copy(wout, w_hbm.at[pl.ds(tg * E * L, E * L)])


def _gate_sc(xg, wgq, bg):
    mesh = plsc.VectorSubcoreMesh(core_axis_name="c", subcore_axis_name="s",
                                  num_cores=NCORE, num_subcores=NSUB)
    fn = functools.partial(
        pl.kernel, mesh=mesh,
        out_type=jax.ShapeDtypeStruct((TG * E * L,), jnp.float32),
        scratch_types=[
            pltpu.VMEM((D * L,), jnp.float32),
            pltpu.VMEM((D * L,), jnp.float32),
            pltpu.VMEM((L,), jnp.float32),
            pltpu.VMEM((L * L,), jnp.float32),
            pltpu.VMEM((E * L,), jnp.float32),
            pltpu.VMEM((E * L,), jnp.float32),
            pltpu.VMEM((E * L,), jnp.float32),
            pltpu.VMEM((E * L,), jnp.float32),
            pltpu.MemorySpace.VMEM_SHARED((4 * E * L,), jnp.float32),
        ],
    )(_gate_body)
    return fn(xg, wgq, bg)


def _moe_body(x_ref, w_ref, W1_ref, b1_ref, W2_ref, b2_ref,
              out_ref, acc_ref):
    e = pl.program_id(0)

    @pl.when(e == 0)
    def _init():
        acc_ref[:] = jnp.zeros_like(acc_ref)

    xb = x_ref[:]
    h = jnp.maximum(
        jnp.dot(xb, W1_ref[0], preferred_element_type=jnp.float32)
        + b1_ref[0, 0, :], 0.0)
    o = jnp.dot(h, W2_ref[0], preferred_element_type=jnp.float32)
    eidx = jax.lax.broadcasted_iota(jnp.int32, (T, E), 1)
    wcol = jnp.sum(jnp.where(eidx == e, w_ref[:], 0.0), axis=1, keepdims=True)
    acc_ref[:] += wcol * o

    @pl.when(e == E - 1)
    def _finish():
        out_ref[:] = acc_ref[:] + jnp.dot(
            w_ref[:], b2_ref[:], preferred_element_type=jnp.float32)


def kernel(x, Wg, bg, W1, b1, W2, b2):
    B, S, _ = x.shape
    xf = x.reshape(T, D)
    b1r = b1.reshape(E, 1, F)
    xg = xf.T.reshape(D, TG, L).transpose(1, 0, 2).reshape(-1)
    wgq = Wg.reshape(D, Q, L).transpose(1, 0, 2).reshape(-1)
    wflat = _gate_sc(xg, wgq, bg)
    w = wflat.reshape(TG, E, L).transpose(0, 2, 1).reshape(T, E)
    out = pl.pallas_call(
        _moe_body,
        grid=(E,),
        in_specs=[
            pl.BlockSpec((T, D), lambda e: (0, 0)),
            pl.BlockSpec((T, E), lambda e: (0, 0)),
            pl.BlockSpec((1, D, F), lambda e: (e, 0, 0)),
            pl.BlockSpec((1, 1, F), lambda e: (e, 0, 0)),
            pl.BlockSpec((1, F, D), lambda e: (e, 0, 0)),
            pl.BlockSpec((E, D), lambda e: (0, 0)),
        ],
        out_specs=pl.BlockSpec((T, D), lambda e: (0, 0)),
        out_shape=jax.ShapeDtypeStruct((T, D), jnp.float32),
        scratch_shapes=[
            pltpu.VMEM((T, D), jnp.float32),
        ],
    )(xf, w, W1, b1r, W2, b2)
    return out.reshape(B, S, D)


# speedup vs baseline: 1.0332x; 1.0332x over previous
import functools

import jax
import jax.numpy as jnp
from jax import lax
from jax.experimental import pallas as pl
from jax.experimental.pallas import tpu as pltpu
from jax.experimental.pallas import tpu_sc as plsc

E = 64
K = 2
D = 768
F = 1536
T = 128
L = 16
NCORE = 2
NSUB = 16
TG = T // L


def _logits_body(x_ref, Wg_ref, bg_ref, out_ref):
    out_ref[:] = lax.dot_general(
        Wg_ref[:], x_ref[:], (((0,), (1,)), ((), ())),
        preferred_element_type=jnp.float32) + bg_ref[:]


def _route_body(lt_hbm, w_hbm, ltv, exv, pv, wout):
    cid = lax.axis_index("c")
    sid = lax.axis_index("s")
    q = sid % 4
    tg = cid * 4 + sid // 4

    @pl.when(q == 0)
    def _route():
        pltpu.sync_copy(lt_hbm, ltv)
        off = tg * L
        mv = ltv[pl.ds(off, L)]
        for ge in range(1, E):
            mv = jnp.maximum(mv, ltv[pl.ds(ge * T + off, L)])
        s = jnp.zeros((L,), jnp.float32)
        for ge in range(E):
            ex = jnp.exp(ltv[pl.ds(ge * T + off, L)] - mv)
            exv[pl.ds(ge * L, L)] = ex
            s = s + ex
        m1 = jnp.full((L,), -1.0, jnp.float32)
        for ge in range(E):
            p = exv[pl.ds(ge * L, L)] / s
            pv[pl.ds(ge * L, L)] = p
            m1 = jnp.maximum(m1, p)
        big = jnp.full((L,), E, jnp.int32)
        i1 = big
        for ge in range(E):
            p = pv[pl.ds(ge * L, L)]
            i1 = jnp.minimum(i1, jnp.where(p == m1, ge, E))
        m2 = jnp.full((L,), -1.0, jnp.float32)
        for ge in range(E):
            p = pv[pl.ds(ge * L, L)]
            m2 = jnp.maximum(m2, jnp.where(i1 == ge, -1.0, p))
        i2 = big
        for ge in range(E):
            p = jnp.where(i1 == ge, -1.0, pv[pl.ds(ge * L, L)])
            i2 = jnp.minimum(i2, jnp.where(p == m2, ge, E))
        denom = m1 + m2
        a1 = m1 / denom
        a2 = m2 / denom
        for ge in range(E):
            wge = (jnp.where(i1 == ge, a1, 0.0)
                   + jnp.where(i2 == ge, a2, 0.0))
            wout[pl.ds(ge * L, L)] = wge
        for ge in range(E):
            pltpu.sync_copy(wout.at[pl.ds(ge * L, L)],
                            w_hbm.at[pl.ds(ge * T + off, L)])


def _route_sc(lt):
    mesh = plsc.VectorSubcoreMesh(core_axis_name="c", subcore_axis_name="s",
                                  num_cores=NCORE, num_subcores=NSUB)
    fn = functools.partial(
        pl.kernel, mesh=mesh,
        out_type=jax.ShapeDtypeStruct((E * T,), jnp.float32),
        scratch_types=[
            pltpu.VMEM((E * T,), jnp.float32),
            pltpu.VMEM((E * L,), jnp.float32),
            pltpu.VMEM((E * L,), jnp.float32),
            pltpu.VMEM((E * L,), jnp.float32),
        ],
    )(_route_body)
    return fn(lt)


def _moe_body(x_ref, wT_ref, W1_ref, b1_ref, W2_ref, b2_ref,
              out_ref, acc_ref):
    e = pl.program_id(0)

    @pl.when(e == 0)
    def _init():
        acc_ref[:] = jnp.zeros_like(acc_ref)

    xb = x_ref[:]
    h = jnp.maximum(
        jnp.dot(xb, W1_ref[0], preferred_element_type=jnp.float32)
        + b1_ref[0, 0, :], 0.0)
    o = jnp.dot(h, W2_ref[0], preferred_element_type=jnp.float32)
    ridx = jax.lax.broadcasted_iota(jnp.int32, (E, 1), 0)
    onehot = (ridx == e).astype(jnp.float32)
    wcol = lax.dot_general(wT_ref[:], onehot, (((0,), (0,)), ((), ())),
                           preferred_element_type=jnp.float32)
    acc_ref[:] += wcol * o

    @pl.when(e == E - 1)
    def _finish():
        out_ref[:] = acc_ref[:] + lax.dot_general(
            wT_ref[:], b2_ref[:], (((0,), (0,)), ((), ())),
            preferred_element_type=jnp.float32)


def kernel(x, Wg, bg, W1, b1, W2, b2):
    B, S, _ = x.shape
    xf = x.reshape(T, D)
    b1r = b1.reshape(E, 1, F)
    lt = pl.pallas_call(
        _logits_body,
        in_specs=[
            pl.BlockSpec((T, D), lambda: (0, 0)),
            pl.BlockSpec((D, E), lambda: (0, 0)),
            pl.BlockSpec((E, 1), lambda: (0, 0)),
        ],
        out_specs=pl.BlockSpec((E, T), lambda: (0, 0)),
        out_shape=jax.ShapeDtypeStruct((E, T), jnp.float32),
    )(xf, Wg, bg.reshape(E, 1))
    wT = _route_sc(lt.reshape(-1)).reshape(E, T)
    out = pl.pallas_call(
        _moe_body,
        grid=(E,),
        in_specs=[
            pl.BlockSpec((T, D), lambda e: (0, 0)),
            pl.BlockSpec((E, T), lambda e: (0, 0)),
            pl.BlockSpec((1, D, F), lambda e: (e, 0, 0)),
            pl.BlockSpec((1, 1, F), lambda e: (e, 0, 0)),
            pl.BlockSpec((1, F, D), lambda e: (e, 0, 0)),
            pl.BlockSpec((E, D), lambda e: (0, 0)),
        ],
        out_specs=pl.BlockSpec((T, D), lambda e: (0, 0)),
        out_shape=jax.ShapeDtypeStruct((T, D), jnp.float32),
        scratch_shapes=[
            pltpu.VMEM((T, D), jnp.float32),
        ],
    )(xf, wT, W1, b1r, W2, b2)
    return out.reshape(B, S, D)


# speedup vs baseline: 1.0517x; 1.0178x over previous
import functools

import jax
import jax.numpy as jnp
from jax import lax
from jax.experimental import pallas as pl
from jax.experimental.pallas import tpu as pltpu
from jax.experimental.pallas import tpu_sc as plsc

E = 64
K = 2
D = 768
F = 1536
T = 128
L = 16
NCORE = 2
NSUB = 16
TG = T // L


def _logits_body(x_ref, Wg_ref, bg_ref, out_ref):
    out_ref[:] = lax.dot_general(
        Wg_ref[:], x_ref[:], (((0,), (1,)), ((), ())),
        preferred_element_type=jnp.float32) + bg_ref[:]


def _route_body(lt_hbm, w_hbm, ltv, exv, pv, wout, sem):
    cid = lax.axis_index("c")
    sid = lax.axis_index("s")
    q = sid % 4
    tg = cid * 4 + sid // 4

    @pl.when(q == 0)
    def _route():
        pltpu.sync_copy(lt_hbm, ltv)
        off = tg * L
        mv = ltv[pl.ds(off, L)]
        for ge in range(1, E):
            mv = jnp.maximum(mv, ltv[pl.ds(ge * T + off, L)])
        s = jnp.zeros((L,), jnp.float32)
        for ge in range(E):
            ex = jnp.exp(ltv[pl.ds(ge * T + off, L)] - mv)
            exv[pl.ds(ge * L, L)] = ex
            s = s + ex
        m1 = jnp.full((L,), -1.0, jnp.float32)
        for ge in range(E):
            p = exv[pl.ds(ge * L, L)] / s
            pv[pl.ds(ge * L, L)] = p
            m1 = jnp.maximum(m1, p)
        big = jnp.full((L,), E, jnp.int32)
        i1 = big
        for ge in range(E):
            p = pv[pl.ds(ge * L, L)]
            i1 = jnp.minimum(i1, jnp.where(p == m1, ge, E))
        m2 = jnp.full((L,), -1.0, jnp.float32)
        for ge in range(E):
            p = pv[pl.ds(ge * L, L)]
            m2 = jnp.maximum(m2, jnp.where(i1 == ge, -1.0, p))
        i2 = big
        for ge in range(E):
            p = jnp.where(i1 == ge, -1.0, pv[pl.ds(ge * L, L)])
            i2 = jnp.minimum(i2, jnp.where(p == m2, ge, E))
        denom = m1 + m2
        a1 = m1 / denom
        a2 = m2 / denom
        for ge in range(E):
            wge = (jnp.where(i1 == ge, a1, 0.0)
                   + jnp.where(i2 == ge, a2, 0.0))
            wout[pl.ds(ge * L, L)] = wge
        descs = [pltpu.make_async_copy(wout.at[pl.ds(ge * L, L)],
                                       w_hbm.at[pl.ds(ge * T + off, L)],
                                       sem)
                 for ge in range(E)]
        for dsc in descs:
            dsc.start()
        for dsc in descs:
            dsc.wait()


def _route_sc(lt):
    mesh = plsc.VectorSubcoreMesh(core_axis_name="c", subcore_axis_name="s",
                                  num_cores=NCORE, num_subcores=NSUB)
    fn = functools.partial(
        pl.kernel, mesh=mesh,
        out_type=jax.ShapeDtypeStruct((E * T,), jnp.float32),
        scratch_types=[
            pltpu.VMEM((E * T,), jnp.float32),
            pltpu.VMEM((E * L,), jnp.float32),
            pltpu.VMEM((E * L,), jnp.float32),
            pltpu.VMEM((E * L,), jnp.float32),
            pltpu.SemaphoreType.DMA,
        ],
    )(_route_body)
    return fn(lt)


def _moe_body(x_ref, wT_ref, W1_ref, b1_ref, W2_ref, b2_ref,
              out_ref, acc_ref):
    e = pl.program_id(0)

    @pl.when(e == 0)
    def _init():
        acc_ref[:] = jnp.zeros_like(acc_ref)

    xb = x_ref[:]
    h = jnp.maximum(
        jnp.dot(xb, W1_ref[0], preferred_element_type=jnp.float32)
        + b1_ref[0, 0, :], 0.0)
    o = jnp.dot(h, W2_ref[0], preferred_element_type=jnp.float32)
    ridx = jax.lax.broadcasted_iota(jnp.int32, (E, 1), 0)
    onehot = (ridx == e).astype(jnp.float32)
    wcol = lax.dot_general(wT_ref[:], onehot, (((0,), (0,)), ((), ())),
                           preferred_element_type=jnp.float32)
    acc_ref[:] += wcol * o

    @pl.when(e == E - 1)
    def _finish():
        out_ref[:] = acc_ref[:] + lax.dot_general(
            wT_ref[:], b2_ref[:], (((0,), (0,)), ((), ())),
            preferred_element_type=jnp.float32)


def kernel(x, Wg, bg, W1, b1, W2, b2):
    B, S, _ = x.shape
    xf = x.reshape(T, D)
    b1r = b1.reshape(E, 1, F)
    lt = pl.pallas_call(
        _logits_body,
        in_specs=[
            pl.BlockSpec((T, D), lambda: (0, 0)),
            pl.BlockSpec((D, E), lambda: (0, 0)),
            pl.BlockSpec((E, 1), lambda: (0, 0)),
        ],
        out_specs=pl.BlockSpec((E, T), lambda: (0, 0)),
        out_shape=jax.ShapeDtypeStruct((E, T), jnp.float32),
    )(xf, Wg, bg.reshape(E, 1))
    wT = _route_sc(lt.reshape(-1)).reshape(E, T)
    out = pl.pallas_call(
        _moe_body,
        grid=(E,),
        in_specs=[
            pl.BlockSpec((T, D), lambda e: (0, 0)),
            pl.BlockSpec((E, T), lambda e: (0, 0)),
            pl.BlockSpec((1, D, F), lambda e: (e, 0, 0)),
            pl.BlockSpec((1, 1, F), lambda e: (e, 0, 0)),
            pl.BlockSpec((1, F, D), lambda e: (e, 0, 0)),
            pl.BlockSpec((E, D), lambda e: (0, 0)),
        ],
        out_specs=pl.BlockSpec((T, D), lambda e: (0, 0)),
        out_shape=jax.ShapeDtypeStruct((T, D), jnp.float32),
        scratch_shapes=[
            pltpu.VMEM((T, D), jnp.float32),
        ],
    )(xf, wT, W1, b1r, W2, b2)
    return out.reshape(B, S, D)
